# Initial kernel scaffold; baseline (speedup 1.0000x reference)
import jax
import jax.numpy as jnp
from jax.experimental import pallas as pl


def kernel(tokens, table):
    raise NotImplementedError("write your pallas kernel here")


# speedup vs baseline: 1.3158x; 1.3158x over previous
import functools
import math

import jax
import jax.numpy as jnp
from jax import lax
from jax.experimental import pallas as pl
from jax.experimental.pallas import tpu as pltpu
from jax.experimental.pallas import tpu_sc as plsc

DIM = 32
SCALE = math.sqrt(32.0)

_NC = 2
_NS = 16
_NW = _NC * _NS

_B = 4096 * 200
_BPW = _B // _NW
_CHUNK = 1600
_NCHUNK = _BPW // _CHUNK


def _emb_kernel(idx_hbm, table_hbm, out_hbm, idx_v, rows_v, sem):
    wid = lax.axis_index("s") * _NC + lax.axis_index("c")
    base = wid * _BPW
    pltpu.sync_copy(idx_hbm.at[pl.ds(base, _BPW)], idx_v)

    def chunk_body(c, carry):
        off = c * _CHUNK
        pltpu.async_copy(
            table_hbm.at[idx_v.at[pl.ds(off, _CHUNK)]], rows_v, sem
        ).wait()

        def row_body(i, carry2):
            rows_v[i, pl.ds(0, 16)] = rows_v[i, pl.ds(0, 16)] * SCALE
            rows_v[i, pl.ds(16, 16)] = rows_v[i, pl.ds(16, 16)] * SCALE
            return carry2

        lax.fori_loop(0, _CHUNK, row_body, 0)
        pltpu.sync_copy(rows_v, out_hbm.at[pl.ds(base + off, _CHUNK)])
        return carry

    lax.fori_loop(0, _NCHUNK, chunk_body, 0)


@functools.partial(
    pl.kernel,
    mesh=plsc.VectorSubcoreMesh(core_axis_name="c", subcore_axis_name="s"),
    out_type=jax.ShapeDtypeStruct((_B, DIM), jnp.float32),
    scratch_types=[
        pltpu.VMEM((_BPW,), jnp.int32),
        pltpu.VMEM((_CHUNK, DIM), jnp.float32),
        pltpu.SemaphoreType.DMA,
    ],
    compiler_params=pltpu.CompilerParams(use_tc_tiling_on_sc=False),
)
def _emb(idx_hbm, table_hbm, out_hbm, idx_v, rows_v, sem):
    _emb_kernel(idx_hbm, table_hbm, out_hbm, idx_v, rows_v, sem)


@jax.jit
def kernel(tokens, table):
    idx = tokens.reshape(-1).astype(jnp.int32)
    out = _emb(idx, table)
    return out.reshape(tokens.shape + (DIM,))


# speedup vs baseline: 1.4749x; 1.1209x over previous
import functools
import math

import jax
import jax.numpy as jnp
from jax import lax
from jax.experimental import pallas as pl
from jax.experimental.pallas import tpu as pltpu
from jax.experimental.pallas import tpu_sc as plsc

DIM = 32
SCALE = math.sqrt(32.0)

_NC = 2
_NS = 16
_NW = _NC * _NS

_B = 4096 * 200
_BPW = _B // _NW
_CHUNK = 1280
_NCHUNK = _BPW // _CHUNK
_UNROLL = 8


def _scale_buf(buf):
    def body(j, carry):
        i0 = j * _UNROLL
        for r in range(_UNROLL):
            buf[i0 + r, pl.ds(0, 16)] = buf[i0 + r, pl.ds(0, 16)] * SCALE
            buf[i0 + r, pl.ds(16, 16)] = buf[i0 + r, pl.ds(16, 16)] * SCALE
        return carry

    lax.fori_loop(0, _CHUNK // _UNROLL, body, 0)


def _emb_kernel(idx_hbm, table_hbm, out_hbm, idx_v, rows0, rows1,
                gsem0, gsem1, ssem0, ssem1):
    rows = (rows0, rows1)
    gsem = (gsem0, gsem1)
    ssem = (ssem0, ssem1)
    wid = lax.axis_index("s") * _NC + lax.axis_index("c")
    base = wid * _BPW
    pltpu.sync_copy(idx_hbm.at[pl.ds(base, _BPW)], idx_v)

    def start_gather(c, b):
        off = c * _CHUNK
        return pltpu.async_copy(
            table_hbm.at[idx_v.at[pl.ds(off, _CHUNK)]], rows[b], gsem[b]
        )

    gathers = {0: start_gather(0, 0)}
    scatters = {}
    for c in range(_NCHUNK):
        b = c % 2
        if c >= 1:
            scatters[c - 1].wait()
        if c + 1 < _NCHUNK:
            gathers[c + 1] = start_gather(c + 1, 1 - b)
        gathers[c].wait()
        _scale_buf(rows[b])
        scatters[c] = pltpu.async_copy(
            rows[b], out_hbm.at[pl.ds(base + c * _CHUNK, _CHUNK)], ssem[b]
        )
    scatters[_NCHUNK - 1].wait()


@functools.partial(
    pl.kernel,
    mesh=plsc.VectorSubcoreMesh(core_axis_name="c", subcore_axis_name="s"),
    out_type=jax.ShapeDtypeStruct((_B, DIM), jnp.float32),
    scratch_types=[
        pltpu.VMEM((_BPW,), jnp.int32),
        pltpu.VMEM((_CHUNK, DIM), jnp.float32),
        pltpu.VMEM((_CHUNK, DIM), jnp.float32),
        pltpu.SemaphoreType.DMA,
        pltpu.SemaphoreType.DMA,
        pltpu.SemaphoreType.DMA,
        pltpu.SemaphoreType.DMA,
    ],
    compiler_params=pltpu.CompilerParams(use_tc_tiling_on_sc=False),
)
def _emb(idx_hbm, table_hbm, out_hbm, idx_v, rows0, rows1,
         gsem0, gsem1, ssem0, ssem1):
    _emb_kernel(idx_hbm, table_hbm, out_hbm, idx_v, rows0, rows1,
                gsem0, gsem1, ssem0, ssem1)


@jax.jit
def kernel(tokens, table):
    idx = tokens.reshape(-1).astype(jnp.int32)
    out = _emb(idx, table)
    return out.reshape(tokens.shape + (DIM,))
